# baseline (speedup 1.0000x reference)
import functools
import jax
import jax.numpy as jnp
from jax import lax
from jax.experimental import pallas as pl
from jax.experimental.pallas import tpu as pltpu
from jax.experimental.pallas import tpu_sc as plsc

N_TASK = 5000
N_AG = 5000
DEG = 64
E = N_AG * DEG
D = 128
NCH = D // 16
NC, NS, L = 2, 16, 16
NW = NC * NS
APW = 160
N_AG_PAD = NW * APW
E_PAD = N_AG_PAD * DEG
EPW = APW * DEG
EPS = 1e-5

_mesh = plsc.VectorSubcoreMesh(
    core_axis_name="c", subcore_axis_name="s", num_cores=NC, num_subcores=NS)


def _worker_id():
    return lax.axis_index("s") * NC + lax.axis_index("c")


def _proj_body(nf_t, nf_a, w1a, w1b, a_out, b_out):
    a_out[...] = jnp.dot(nf_t[...], w1a[...], preferred_element_type=jnp.float32)
    b_out[0:N_AG, :] = jnp.dot(nf_a[...], w1b[...],
                               preferred_element_type=jnp.float32)


_proj = pl.pallas_call(
    _proj_body,
    out_shape=[
        jax.ShapeDtypeStruct((N_TASK, D), jnp.float32),
        jax.ShapeDtypeStruct((N_AG_PAD, D), jnp.float32),
    ],
)


@functools.partial(
    pl.kernel,
    out_type=jax.ShapeDtypeStruct((NW * 2 * D,), jnp.float32),
    compiler_params=pltpu.CompilerParams(needs_layout_passes=False),
    mesh=_mesh,
    scratch_types=[
        pltpu.VMEM((EPW,), jnp.int32),
        pltpu.VMEM((APW, D), jnp.float32),
        pltpu.VMEM((2, 4 * DEG, D), jnp.float32),
        pltpu.VMEM((2 * D,), jnp.float32),
        pltpu.SemaphoreType.DMA,
        pltpu.SemaphoreType.DMA,
        pltpu.SemaphoreType.DMA,
        pltpu.SemaphoreType.DMA,
    ],
)
def _pass1(a_hbm, b_hbm, src_hbm, out_hbm, src_v, b_v, g_v, o_v,
           sem0, sem1, sem2, sem3):
    w = _worker_id()
    a0 = w * APW
    pltpu.sync_copy(src_hbm.at[pl.ds(a0 * DEG, EPW)], src_v)
    pltpu.sync_copy(b_hbm.at[pl.ds(a0, APW)], b_v)
    nv = jnp.minimum(APW, N_TASK - a0)
    zero = jnp.zeros((L,), jnp.float32)
    sems = ((sem0, sem2), (sem1, sem3))
    CK = 4 * DEG
    HK = CK // 2
    ncks = nv // 4
    pltpu.async_copy(a_hbm.at[src_v.at[pl.ds(0, HK)]],
                     g_v.at[0, pl.ds(0, HK)], sems[0][0])
    pltpu.async_copy(a_hbm.at[src_v.at[pl.ds(HK, HK)]],
                     g_v.at[0, pl.ds(HK, HK)], sems[0][1])

    def inner(i, carry):
        par, c, sums = carry
        b = [b_v[c * 4 + i, pl.ds(16 * j, L)] for j in range(NCH)]
        sA = [zero] * NCH
        sA2 = [zero] * NCH
        for e in range(DEG):
            for j in range(NCH):
                a_vec = g_v[par, i * DEG + e, pl.ds(16 * j, L)]
                sA[j] = sA[j] + a_vec
                sA2[j] = sA2[j] + a_vec * a_vec
        out = []
        for j in range(NCH):
            out.append(sums[j] + sA[j] + 64.0 * b[j])
        for j in range(NCH):
            out.append(sums[NCH + j] + sA2[j] + 2.0 * b[j] * sA[j]
                       + 64.0 * b[j] * b[j])
        return par, c, tuple(out)

    def body(i2, carry):
        for par in (0, 1):
            c = i2 * 2 + par
            for h in (0, 1):
                pltpu.make_async_copy(
                    a_hbm.at[pl.ds(0, HK)],
                    g_v.at[par, pl.ds(h * HK, HK)], sems[par][h]).wait()
            nxt = jnp.minimum(c + 1, ncks - 1)
            for h in (0, 1):
                pltpu.async_copy(
                    a_hbm.at[src_v.at[pl.ds(nxt * CK + h * HK, HK)]],
                    g_v.at[par ^ 1, pl.ds(h * HK, HK)], sems[par ^ 1][h])
            _, _, carry = lax.fori_loop(
                0, 4, inner, (jnp.int32(par), jnp.int32(c), carry))
        return carry

    res = lax.fori_loop(0, ncks // 2, body, tuple([zero] * (2 * NCH)))
    for h in (0, 1):
        pltpu.make_async_copy(a_hbm.at[pl.ds(0, HK)],
                              g_v.at[0, pl.ds(h * HK, HK)], sems[0][h]).wait()
    for j in range(2 * NCH):
        o_v[pl.ds(16 * j, L)] = res[j]
    pltpu.sync_copy(o_v, out_hbm.at[pl.ds(w * 2 * D, 2 * D)])


def _fold_body(p1, a_ref, b_ref, g1, b1, w2_ref, nt_ref,
               abar_o, bbar_o, p_o, q_o, c2_o, f_o):
    part = p1[...]
    sums = jnp.sum(part[:, :D], axis=0)
    sqs = jnp.sum(part[:, D:], axis=0)
    mean = sums / E
    var = sqs / E - mean * mean
    inv = g1[...] * lax.rsqrt(var + EPS)
    bias = b1[...] - mean * inv
    abar = a_ref[...] * inv
    bbar = b_ref[0:N_AG, :] * inv + bias
    abar_o[...] = abar
    bbar_o[0:N_AG, :] = bbar
    w2 = w2_ref[...][:, 0]
    p_o[...] = jnp.dot(abar, 0.505 * w2, preferred_element_type=jnp.float32)
    q_o[0:N_AG] = jnp.dot(bbar, 0.505 * w2, preferred_element_type=jnp.float32)
    c2_o[...] = 0.495 * w2
    f_o[...] = (nt_ref[...] == 3).astype(jnp.float32)


_fold = pl.pallas_call(
    _fold_body,
    out_shape=[
        jax.ShapeDtypeStruct((N_TASK, D), jnp.float32),
        jax.ShapeDtypeStruct((N_AG_PAD, D), jnp.float32),
        jax.ShapeDtypeStruct((N_TASK,), jnp.float32),
        jax.ShapeDtypeStruct((N_AG_PAD,), jnp.float32),
        jax.ShapeDtypeStruct((D,), jnp.float32),
        jax.ShapeDtypeStruct((N_TASK,), jnp.float32),
    ],
)


@functools.partial(
    pl.kernel,
    out_type=[
        jax.ShapeDtypeStruct((E_PAD,), jnp.float32),
        jax.ShapeDtypeStruct((E_PAD,), jnp.float32),
        jax.ShapeDtypeStruct((NW * 16,), jnp.float32),
    ],
    compiler_params=pltpu.CompilerParams(needs_layout_passes=False),
    mesh=_mesh,
    scratch_types=[
        pltpu.VMEM((EPW,), jnp.int32),
        pltpu.VMEM((APW, D), jnp.float32),
        pltpu.VMEM((2, 4 * DEG, D), jnp.float32),
        pltpu.VMEM((N_TASK,), jnp.float32),
        pltpu.VMEM((N_TASK,), jnp.float32),
        pltpu.VMEM((APW,), jnp.float32),
        pltpu.VMEM((D,), jnp.float32),
        pltpu.VMEM((EPW,), jnp.float32),
        pltpu.VMEM((EPW,), jnp.float32),
        pltpu.VMEM((16,), jnp.float32),
        pltpu.SemaphoreType.DMA,
        pltpu.SemaphoreType.DMA,
        pltpu.SemaphoreType.DMA,
        pltpu.SemaphoreType.DMA,
    ],
)
def _pass2(abar_hbm, bbar_hbm, src_hbm, p_hbm, f_hbm, q_hbm, c2_hbm,
           s_hbm, fm_hbm, p2_hbm,
           src_v, b_v, g_v, p_v, f_v, q_v, c2_v, sbuf, fbuf, o_v,
           sem0, sem1, sem2, sem3):
    w = _worker_id()
    a0 = w * APW
    pltpu.sync_copy(src_hbm.at[pl.ds(a0 * DEG, EPW)], src_v)
    pltpu.sync_copy(bbar_hbm.at[pl.ds(a0, APW)], b_v)
    pltpu.sync_copy(p_hbm, p_v)
    pltpu.sync_copy(f_hbm, f_v)
    pltpu.sync_copy(q_hbm.at[pl.ds(a0, APW)], q_v)
    pltpu.sync_copy(c2_hbm, c2_v)
    nv = jnp.minimum(APW, N_TASK - a0)
    zero = jnp.zeros((L,), jnp.float32)
    iota = lax.iota(jnp.int32, L)
    c2 = [c2_v[pl.ds(16 * j, L)] for j in range(NCH)]

    sems = ((sem0, sem2), (sem1, sem3))
    CK = 4 * DEG
    HK = CK // 2
    ncks = nv // 4
    pltpu.async_copy(abar_hbm.at[src_v.at[pl.ds(0, HK)]],
                     g_v.at[0, pl.ds(0, HK)], sems[0][0])
    pltpu.async_copy(abar_hbm.at[src_v.at[pl.ds(HK, HK)]],
                     g_v.at[0, pl.ds(HK, HK)], sems[0][1])

    def inner(ii, carry):
        par, c, ss, ss2 = carry
        i = c * 4 + ii
        b = [b_v[i, pl.ds(16 * j, L)] for j in range(NCH)]
        qs = plsc.load_gather(q_v, [jnp.full((L,), i, jnp.int32)])
        for grp in range(DEG // L):
            sv = src_v[pl.ds(i * DEG + grp * L, L)]
            parts = [zero, zero, zero, zero]
            for k in range(L):
                e = grp * L + k
                acc0 = zero
                acc1 = zero
                for j in range(NCH):
                    u = g_v[par, ii * DEG + e, pl.ds(16 * j, L)] + b[j]
                    if j % 2 == 0:
                        acc0 = acc0 + c2[j] * jnp.abs(u)
                    else:
                        acc1 = acc1 + c2[j] * jnp.abs(u)
                parts[k % 4] = jnp.where(
                    iota == k, jnp.sum(acc0 + acc1), parts[k % 4])
            svec = (parts[0] + parts[1]) + (parts[2] + parts[3])
            svec = svec + plsc.load_gather(p_v, [sv]) + qs
            fg = plsc.load_gather(f_v, [sv])
            ss = ss + svec
            ss2 = ss2 + svec * svec
            sbuf[pl.ds(i * DEG + grp * L, L)] = svec
            fbuf[pl.ds(i * DEG + grp * L, L)] = fg
        return par, c, ss, ss2

    def body(i2, carry):
        for par in (0, 1):
            c = i2 * 2 + par
            for h in (0, 1):
                pltpu.make_async_copy(
                    abar_hbm.at[pl.ds(0, HK)],
                    g_v.at[par, pl.ds(h * HK, HK)], sems[par][h]).wait()
            nxt = jnp.minimum(c + 1, ncks - 1)
            for h in (0, 1):
                pltpu.async_copy(
                    abar_hbm.at[src_v.at[pl.ds(nxt * CK + h * HK, HK)]],
                    g_v.at[par ^ 1, pl.ds(h * HK, HK)], sems[par ^ 1][h])
            _, _, ss, ss2 = lax.fori_loop(
                0, 4, inner, (jnp.int32(par), jnp.int32(c)) + carry)
            carry = (ss, ss2)
        return carry

    ss, ss2 = lax.fori_loop(0, ncks // 2, body, (zero, zero))
    for h in (0, 1):
        pltpu.make_async_copy(abar_hbm.at[pl.ds(0, HK)],
                              g_v.at[0, pl.ds(h * HK, HK)], sems[0][h]).wait()
    ovec = jnp.where(iota == 0, jnp.sum(ss), 0.0)
    ovec = jnp.where(iota == 1, jnp.sum(ss2), ovec)
    o_v[...] = ovec
    pltpu.sync_copy(o_v, p2_hbm.at[pl.ds(w * 16, 16)])
    pltpu.sync_copy(sbuf, s_hbm.at[pl.ds(a0 * DEG, EPW)])
    pltpu.sync_copy(fbuf, fm_hbm.at[pl.ds(a0 * DEG, EPW)])


def _final_body(s_ref, fm_ref, p2_ref, g2, b2, out_ref):
    p2 = p2_ref[...]
    ssum = jnp.sum(p2[:, 0])
    ssq = jnp.sum(p2[:, 1])
    mean = ssum / E
    var = ssq / E - mean * mean
    inv = g2[...] * lax.rsqrt(var + EPS)
    bias = b2[...] - mean * inv
    vals = s_ref[0:N_AG, :] * inv + bias
    out_ref[...] = jnp.where(fm_ref[0:N_AG, :] > 0.5, -jnp.inf, vals)


_final = pl.pallas_call(
    _final_body,
    out_shape=jax.ShapeDtypeStruct((N_AG, DEG), jnp.float32),
)


@jax.jit
def kernel(nf, edge_index, node_type, W1, gamma1, beta1, W2, gamma2, beta2):
    nf_t = nf[:N_TASK]
    nf_a = nf[N_TASK:]
    w1a = W1[:D]
    w1b = W1[D:]
    src = edge_index[0].astype(jnp.int32)
    src_pad = jnp.pad(src, (0, E_PAD - E))

    A, B = _proj(nf_t, nf_a, w1a, w1b)
    part1 = _pass1(A, B, src_pad).reshape(NW, 2 * D)

    abar, bbar, p, q, c2, fnode = _fold(
        part1, A, B, gamma1, beta1, W2, node_type[:N_TASK].astype(jnp.int32))

    s_pad, fm_pad, part2 = _pass2(abar, bbar, src_pad, p, fnode, q, c2)
    part2 = part2.reshape(NW, 16)
    s = s_pad.reshape(N_AG_PAD, DEG)
    fm = fm_pad.reshape(N_AG_PAD, DEG)
    return _final(s, fm, part2, gamma2, beta2)


# speedup vs baseline: 1.0099x; 1.0099x over previous
import functools
import jax
import jax.numpy as jnp
from jax import lax
from jax.experimental import pallas as pl
from jax.experimental.pallas import tpu as pltpu
from jax.experimental.pallas import tpu_sc as plsc

N_TASK = 5000
N_AG = 5000
DEG = 64
E = N_AG * DEG
D = 128
NCH = D // 16
NC, NS, L = 2, 16, 16
NW = NC * NS
APW = 160
N_AG_PAD = NW * APW
E_PAD = N_AG_PAD * DEG
EPW = APW * DEG
EPS = 1e-5

_mesh = plsc.VectorSubcoreMesh(
    core_axis_name="c", subcore_axis_name="s", num_cores=NC, num_subcores=NS)


def _worker_id():
    return lax.axis_index("s") * NC + lax.axis_index("c")


def _proj_body(nf_t, nf_a, w1a, w1b, a_out, b_out):
    a_out[...] = jnp.dot(nf_t[...], w1a[...], preferred_element_type=jnp.float32)
    b_out[0:N_AG, :] = jnp.dot(nf_a[...], w1b[...],
                               preferred_element_type=jnp.float32)


_proj = pl.pallas_call(
    _proj_body,
    out_shape=[
        jax.ShapeDtypeStruct((N_TASK, D), jnp.float32),
        jax.ShapeDtypeStruct((N_AG_PAD, D), jnp.float32),
    ],
)


@functools.partial(
    pl.kernel,
    out_type=jax.ShapeDtypeStruct((NW * 2 * D,), jnp.float32),
    compiler_params=pltpu.CompilerParams(needs_layout_passes=False),
    mesh=_mesh,
    scratch_types=[
        pltpu.VMEM((EPW,), jnp.int32),
        pltpu.VMEM((APW, D), jnp.float32),
        pltpu.VMEM((2, 4 * DEG, D), jnp.float32),
        pltpu.VMEM((2 * D,), jnp.float32),
        pltpu.SemaphoreType.DMA,
        pltpu.SemaphoreType.DMA,
        pltpu.SemaphoreType.DMA,
        pltpu.SemaphoreType.DMA,
    ],
)
def _pass1(a_hbm, b_hbm, src_hbm, out_hbm, src_v, b_v, g_v, o_v,
           sem0, sem1, sem2, sem3):
    w = _worker_id()
    a0 = w * APW
    pltpu.sync_copy(src_hbm.at[pl.ds(a0 * DEG, EPW)], src_v)
    pltpu.sync_copy(b_hbm.at[pl.ds(a0, APW)], b_v)
    nv = jnp.minimum(APW, N_TASK - a0)
    zero = jnp.zeros((L,), jnp.float32)
    sems = ((sem0, sem2), (sem1, sem3))
    CK = 4 * DEG
    HK = CK // 2
    ncks = nv // 4
    pltpu.async_copy(a_hbm.at[src_v.at[pl.ds(0, HK)]],
                     g_v.at[0, pl.ds(0, HK)], sems[0][0])
    pltpu.async_copy(a_hbm.at[src_v.at[pl.ds(HK, HK)]],
                     g_v.at[0, pl.ds(HK, HK)], sems[0][1])

    def inner(i, carry):
        par, c, sums = carry
        b = [b_v[c * 4 + i, pl.ds(16 * j, L)] for j in range(NCH)]
        sA = [zero] * NCH
        sA2 = [zero] * NCH
        for e in range(DEG):
            for j in range(NCH):
                a_vec = g_v[par, i * DEG + e, pl.ds(16 * j, L)]
                sA[j] = sA[j] + a_vec
                sA2[j] = sA2[j] + a_vec * a_vec
        out = []
        for j in range(NCH):
            out.append(sums[j] + sA[j] + 64.0 * b[j])
        for j in range(NCH):
            out.append(sums[NCH + j] + sA2[j] + 2.0 * b[j] * sA[j]
                       + 64.0 * b[j] * b[j])
        return par, c, tuple(out)

    def body(i2, carry):
        for par in (0, 1):
            c = i2 * 2 + par
            for h in (0, 1):
                pltpu.make_async_copy(
                    a_hbm.at[pl.ds(0, HK)],
                    g_v.at[par, pl.ds(h * HK, HK)], sems[par][h]).wait()
            nxt = jnp.minimum(c + 1, ncks - 1)
            for h in (0, 1):
                pltpu.async_copy(
                    a_hbm.at[src_v.at[pl.ds(nxt * CK + h * HK, HK)]],
                    g_v.at[par ^ 1, pl.ds(h * HK, HK)], sems[par ^ 1][h])
            _, _, carry = lax.fori_loop(
                0, 4, inner, (jnp.int32(par), jnp.int32(c), carry))
        return carry

    res = lax.fori_loop(0, ncks // 2, body, tuple([zero] * (2 * NCH)))
    for h in (0, 1):
        pltpu.make_async_copy(a_hbm.at[pl.ds(0, HK)],
                              g_v.at[0, pl.ds(h * HK, HK)], sems[0][h]).wait()
    for j in range(2 * NCH):
        o_v[pl.ds(16 * j, L)] = res[j]
    pltpu.sync_copy(o_v, out_hbm.at[pl.ds(w * 2 * D, 2 * D)])


def _fold_body(p1, a_ref, b_ref, g1, b1, w2_ref, nt_ref,
               abar_o, bbar_o, p_o, q_o, c2_o, f_o):
    part = p1[...]
    sums = jnp.sum(part[:, :D], axis=0)
    sqs = jnp.sum(part[:, D:], axis=0)
    mean = sums / E
    var = sqs / E - mean * mean
    inv = g1[...] * lax.rsqrt(var + EPS)
    bias = b1[...] - mean * inv
    abar = a_ref[...] * inv
    bbar = b_ref[0:N_AG, :] * inv + bias
    abar_o[...] = abar
    bbar_o[0:N_AG, :] = bbar
    w2 = w2_ref[...][:, 0]
    p_o[...] = jnp.dot(abar, 0.505 * w2, preferred_element_type=jnp.float32)
    q_o[0:N_AG] = jnp.dot(bbar, 0.505 * w2, preferred_element_type=jnp.float32)
    c2_o[...] = 0.495 * w2
    f_o[...] = (nt_ref[...] == 3).astype(jnp.float32)


_fold = pl.pallas_call(
    _fold_body,
    out_shape=[
        jax.ShapeDtypeStruct((N_TASK, D), jnp.float32),
        jax.ShapeDtypeStruct((N_AG_PAD, D), jnp.float32),
        jax.ShapeDtypeStruct((N_TASK,), jnp.float32),
        jax.ShapeDtypeStruct((N_AG_PAD,), jnp.float32),
        jax.ShapeDtypeStruct((D,), jnp.float32),
        jax.ShapeDtypeStruct((N_TASK,), jnp.float32),
    ],
)


@functools.partial(
    pl.kernel,
    out_type=[
        jax.ShapeDtypeStruct((E_PAD,), jnp.float32),
        jax.ShapeDtypeStruct((E_PAD,), jnp.float32),
        jax.ShapeDtypeStruct((NW * 16,), jnp.float32),
    ],
    compiler_params=pltpu.CompilerParams(needs_layout_passes=False),
    mesh=_mesh,
    scratch_types=[
        pltpu.VMEM((EPW,), jnp.int32),
        pltpu.VMEM((APW, D), jnp.float32),
        pltpu.VMEM((2, 4 * DEG, D), jnp.float32),
        pltpu.VMEM((N_TASK,), jnp.float32),
        pltpu.VMEM((N_TASK,), jnp.float32),
        pltpu.VMEM((APW,), jnp.float32),
        pltpu.VMEM((D,), jnp.float32),
        pltpu.VMEM((EPW,), jnp.float32),
        pltpu.VMEM((EPW,), jnp.float32),
        pltpu.VMEM((16,), jnp.float32),
        pltpu.SemaphoreType.DMA,
        pltpu.SemaphoreType.DMA,
        pltpu.SemaphoreType.DMA,
        pltpu.SemaphoreType.DMA,
    ],
)
def _pass2(abar_hbm, bbar_hbm, src_hbm, p_hbm, f_hbm, q_hbm, c2_hbm,
           s_hbm, fm_hbm, p2_hbm,
           src_v, b_v, g_v, p_v, f_v, q_v, c2_v, sbuf, fbuf, o_v,
           sem0, sem1, sem2, sem3):
    w = _worker_id()
    a0 = w * APW
    pltpu.sync_copy(src_hbm.at[pl.ds(a0 * DEG, EPW)], src_v)
    pltpu.sync_copy(bbar_hbm.at[pl.ds(a0, APW)], b_v)
    pltpu.sync_copy(p_hbm, p_v)
    pltpu.sync_copy(f_hbm, f_v)
    pltpu.sync_copy(q_hbm.at[pl.ds(a0, APW)], q_v)
    pltpu.sync_copy(c2_hbm, c2_v)
    nv = jnp.minimum(APW, N_TASK - a0)
    zero = jnp.zeros((L,), jnp.float32)
    iota = lax.iota(jnp.int32, L)
    c2 = [c2_v[pl.ds(16 * j, L)] for j in range(NCH)]

    sems = ((sem0, sem2), (sem1, sem3))
    CK = 4 * DEG
    HK = CK // 2
    ncks = nv // 4
    pltpu.async_copy(abar_hbm.at[src_v.at[pl.ds(0, HK)]],
                     g_v.at[0, pl.ds(0, HK)], sems[0][0])
    pltpu.async_copy(abar_hbm.at[src_v.at[pl.ds(HK, HK)]],
                     g_v.at[0, pl.ds(HK, HK)], sems[0][1])

    def inner(ii, carry):
        par, c, ss, ss2 = carry
        i = c * 4 + ii
        b = [b_v[i, pl.ds(16 * j, L)] for j in range(NCH)]
        qs = plsc.load_gather(q_v, [jnp.full((L,), i, jnp.int32)])
        for grp in range(DEG // L):
            sv = src_v[pl.ds(i * DEG + grp * L, L)]
            svec = zero
            for k in range(L):
                e = grp * L + k
                acc = zero
                for j in range(NCH):
                    u = g_v[par, ii * DEG + e, pl.ds(16 * j, L)] + b[j]
                    acc = acc + c2[j] * jnp.abs(u)
                svec = jnp.where(iota == k, jnp.sum(acc), svec)
            svec = svec + plsc.load_gather(p_v, [sv]) + qs
            fg = plsc.load_gather(f_v, [sv])
            ss = ss + svec
            ss2 = ss2 + svec * svec
            sbuf[pl.ds(i * DEG + grp * L, L)] = svec
            fbuf[pl.ds(i * DEG + grp * L, L)] = fg
        return par, c, ss, ss2

    def body(i2, carry):
        for par in (0, 1):
            c = i2 * 2 + par
            for h in (0, 1):
                pltpu.make_async_copy(
                    abar_hbm.at[pl.ds(0, HK)],
                    g_v.at[par, pl.ds(h * HK, HK)], sems[par][h]).wait()
            nxt = jnp.minimum(c + 1, ncks - 1)
            for h in (0, 1):
                pltpu.async_copy(
                    abar_hbm.at[src_v.at[pl.ds(nxt * CK + h * HK, HK)]],
                    g_v.at[par ^ 1, pl.ds(h * HK, HK)], sems[par ^ 1][h])
            _, _, ss, ss2 = lax.fori_loop(
                0, 4, inner, (jnp.int32(par), jnp.int32(c)) + carry)
            carry = (ss, ss2)
        return carry

    ss, ss2 = lax.fori_loop(0, ncks // 2, body, (zero, zero))
    for h in (0, 1):
        pltpu.make_async_copy(abar_hbm.at[pl.ds(0, HK)],
                              g_v.at[0, pl.ds(h * HK, HK)], sems[0][h]).wait()
    ovec = jnp.where(iota == 0, jnp.sum(ss), 0.0)
    ovec = jnp.where(iota == 1, jnp.sum(ss2), ovec)
    o_v[...] = ovec
    pltpu.sync_copy(o_v, p2_hbm.at[pl.ds(w * 16, 16)])
    pltpu.sync_copy(sbuf, s_hbm.at[pl.ds(a0 * DEG, EPW)])
    pltpu.sync_copy(fbuf, fm_hbm.at[pl.ds(a0 * DEG, EPW)])


def _final_body(s_ref, fm_ref, p2_ref, g2, b2, out_ref):
    p2 = p2_ref[...]
    ssum = jnp.sum(p2[:, 0])
    ssq = jnp.sum(p2[:, 1])
    mean = ssum / E
    var = ssq / E - mean * mean
    inv = g2[...] * lax.rsqrt(var + EPS)
    bias = b2[...] - mean * inv
    vals = s_ref[0:N_AG, :] * inv + bias
    out_ref[...] = jnp.where(fm_ref[0:N_AG, :] > 0.5, -jnp.inf, vals)


_final = pl.pallas_call(
    _final_body,
    out_shape=jax.ShapeDtypeStruct((N_AG, DEG), jnp.float32),
)


@jax.jit
def kernel(nf, edge_index, node_type, W1, gamma1, beta1, W2, gamma2, beta2):
    nf_t = nf[:N_TASK]
    nf_a = nf[N_TASK:]
    w1a = W1[:D]
    w1b = W1[D:]
    src = edge_index[0].astype(jnp.int32)
    src_pad = jnp.pad(src, (0, E_PAD - E))

    A, B = _proj(nf_t, nf_a, w1a, w1b)
    part1 = _pass1(A, B, src_pad).reshape(NW, 2 * D)

    abar, bbar, p, q, c2, fnode = _fold(
        part1, A, B, gamma1, beta1, W2, node_type[:N_TASK].astype(jnp.int32))

    s_pad, fm_pad, part2 = _pass2(abar, bbar, src_pad, p, fnode, q, c2)
    part2 = part2.reshape(NW, 16)
    s = s_pad.reshape(N_AG_PAD, DEG)
    fm = fm_pad.reshape(N_AG_PAD, DEG)
    return _final(s, fm, part2, gamma2, beta2)


# speedup vs baseline: 1.0282x; 1.0181x over previous
import functools
import jax
import jax.numpy as jnp
from jax import lax
from jax.experimental import pallas as pl
from jax.experimental.pallas import tpu as pltpu
from jax.experimental.pallas import tpu_sc as plsc

N_TASK = 5000
N_AG = 5000
DEG = 64
E = N_AG * DEG
D = 128
NCH = D // 16
NC, NS, L = 2, 16, 16
NW = NC * NS
APW = 160
N_AG_PAD = NW * APW
E_PAD = N_AG_PAD * DEG
EPW = APW * DEG
EPS = 1e-5

_mesh = plsc.VectorSubcoreMesh(
    core_axis_name="c", subcore_axis_name="s", num_cores=NC, num_subcores=NS)


def _worker_id():
    return lax.axis_index("s") * NC + lax.axis_index("c")


def _proj_body(nf_t, nf_a, w1a, w1b, a_out, b_out):
    a_out[...] = jnp.dot(nf_t[...], w1a[...], preferred_element_type=jnp.float32)
    b_out[0:N_AG, :] = jnp.dot(nf_a[...], w1b[...],
                               preferred_element_type=jnp.float32)


_proj = pl.pallas_call(
    _proj_body,
    out_shape=[
        jax.ShapeDtypeStruct((N_TASK, D), jnp.float32),
        jax.ShapeDtypeStruct((N_AG_PAD, D), jnp.float32),
    ],
)


@functools.partial(
    pl.kernel,
    out_type=jax.ShapeDtypeStruct((NW * 2 * D,), jnp.float32),
    compiler_params=pltpu.CompilerParams(needs_layout_passes=False),
    mesh=_mesh,
    scratch_types=[
        pltpu.VMEM((EPW,), jnp.int32),
        pltpu.VMEM((APW, D), jnp.float32),
        pltpu.VMEM((2, 5 * DEG, D), jnp.float32),
        pltpu.VMEM((2 * D,), jnp.float32),
        pltpu.SemaphoreType.DMA,
        pltpu.SemaphoreType.DMA,
        pltpu.SemaphoreType.DMA,
        pltpu.SemaphoreType.DMA,
    ],
)
def _pass1(a_hbm, b_hbm, src_hbm, out_hbm, src_v, b_v, g_v, o_v,
           sem0, sem1, sem2, sem3):
    w = _worker_id()
    a0 = w * APW
    pltpu.sync_copy(src_hbm.at[pl.ds(a0 * DEG, EPW)], src_v)
    pltpu.sync_copy(b_hbm.at[pl.ds(a0, APW)], b_v)
    nv = jnp.minimum(APW, N_TASK - a0)
    zero = jnp.zeros((L,), jnp.float32)
    sems = ((sem0, sem2), (sem1, sem3))
    NAC = 5
    CK = NAC * DEG
    HK = CK // 2
    ncks = nv // NAC
    pltpu.async_copy(a_hbm.at[src_v.at[pl.ds(0, HK)]],
                     g_v.at[0, pl.ds(0, HK)], sems[0][0])
    pltpu.async_copy(a_hbm.at[src_v.at[pl.ds(HK, HK)]],
                     g_v.at[0, pl.ds(HK, HK)], sems[0][1])

    def inner(i, carry):
        par, c, sums = carry
        b = [b_v[c * NAC + i, pl.ds(16 * j, L)] for j in range(NCH)]
        sA = [zero] * NCH
        sA2 = [zero] * NCH
        for e in range(DEG):
            for j in range(NCH):
                a_vec = g_v[par, i * DEG + e, pl.ds(16 * j, L)]
                sA[j] = sA[j] + a_vec
                sA2[j] = sA2[j] + a_vec * a_vec
        out = []
        for j in range(NCH):
            out.append(sums[j] + sA[j] + 64.0 * b[j])
        for j in range(NCH):
            out.append(sums[NCH + j] + sA2[j] + 2.0 * b[j] * sA[j]
                       + 64.0 * b[j] * b[j])
        return par, c, tuple(out)

    def body(i2, carry):
        for par in (0, 1):
            c = i2 * 2 + par
            for h in (0, 1):
                pltpu.make_async_copy(
                    a_hbm.at[pl.ds(0, HK)],
                    g_v.at[par, pl.ds(h * HK, HK)], sems[par][h]).wait()
            nxt = jnp.minimum(c + 1, ncks - 1)
            for h in (0, 1):
                pltpu.async_copy(
                    a_hbm.at[src_v.at[pl.ds(nxt * CK + h * HK, HK)]],
                    g_v.at[par ^ 1, pl.ds(h * HK, HK)], sems[par ^ 1][h])
            _, _, carry = lax.fori_loop(
                0, NAC, inner, (jnp.int32(par), jnp.int32(c), carry))
        return carry

    res = lax.fori_loop(0, ncks // 2, body, tuple([zero] * (2 * NCH)))
    for h in (0, 1):
        pltpu.make_async_copy(a_hbm.at[pl.ds(0, HK)],
                              g_v.at[0, pl.ds(h * HK, HK)], sems[0][h]).wait()
    for j in range(2 * NCH):
        o_v[pl.ds(16 * j, L)] = res[j]
    pltpu.sync_copy(o_v, out_hbm.at[pl.ds(w * 2 * D, 2 * D)])


def _fold_body(p1, a_ref, b_ref, g1, b1, w2_ref, nt_ref,
               abar_o, bbar_o, p_o, q_o, c2_o, f_o):
    part = p1[...]
    sums = jnp.sum(part[:, :D], axis=0)
    sqs = jnp.sum(part[:, D:], axis=0)
    mean = sums / E
    var = sqs / E - mean * mean
    inv = g1[...] * lax.rsqrt(var + EPS)
    bias = b1[...] - mean * inv
    abar = a_ref[...] * inv
    bbar = b_ref[0:N_AG, :] * inv + bias
    abar_o[...] = abar
    bbar_o[0:N_AG, :] = bbar
    w2 = w2_ref[...][:, 0]
    p_o[...] = jnp.dot(abar, 0.505 * w2, preferred_element_type=jnp.float32)
    q_o[0:N_AG] = jnp.dot(bbar, 0.505 * w2, preferred_element_type=jnp.float32)
    c2_o[...] = 0.495 * w2
    f_o[...] = (nt_ref[...] == 3).astype(jnp.float32)


_fold = pl.pallas_call(
    _fold_body,
    out_shape=[
        jax.ShapeDtypeStruct((N_TASK, D), jnp.float32),
        jax.ShapeDtypeStruct((N_AG_PAD, D), jnp.float32),
        jax.ShapeDtypeStruct((N_TASK,), jnp.float32),
        jax.ShapeDtypeStruct((N_AG_PAD,), jnp.float32),
        jax.ShapeDtypeStruct((D,), jnp.float32),
        jax.ShapeDtypeStruct((N_TASK,), jnp.float32),
    ],
)


@functools.partial(
    pl.kernel,
    out_type=[
        jax.ShapeDtypeStruct((E_PAD,), jnp.float32),
        jax.ShapeDtypeStruct((NW * 16,), jnp.float32),
    ],
    compiler_params=pltpu.CompilerParams(needs_layout_passes=False),
    mesh=_mesh,
    scratch_types=[
        pltpu.VMEM((EPW,), jnp.int32),
        pltpu.VMEM((APW, D), jnp.float32),
        pltpu.VMEM((2, 4 * DEG, D), jnp.float32),
        pltpu.VMEM((N_TASK,), jnp.float32),
        pltpu.VMEM((N_TASK,), jnp.float32),
        pltpu.VMEM((APW,), jnp.float32),
        pltpu.VMEM((D,), jnp.float32),
        pltpu.VMEM((EPW,), jnp.float32),
        pltpu.VMEM((16,), jnp.float32),
        pltpu.SemaphoreType.DMA,
        pltpu.SemaphoreType.DMA,
        pltpu.SemaphoreType.DMA,
        pltpu.SemaphoreType.DMA,
    ],
)
def _pass2(abar_hbm, bbar_hbm, src_hbm, p_hbm, f_hbm, q_hbm, c2_hbm,
           s_hbm, p2_hbm,
           src_v, b_v, g_v, p_v, f_v, q_v, c2_v, sbuf, o_v,
           sem0, sem1, sem2, sem3):
    w = _worker_id()
    a0 = w * APW
    pltpu.sync_copy(src_hbm.at[pl.ds(a0 * DEG, EPW)], src_v)
    pltpu.sync_copy(bbar_hbm.at[pl.ds(a0, APW)], b_v)
    pltpu.sync_copy(p_hbm, p_v)
    pltpu.sync_copy(f_hbm, f_v)
    pltpu.sync_copy(q_hbm.at[pl.ds(a0, APW)], q_v)
    pltpu.sync_copy(c2_hbm, c2_v)
    nv = jnp.minimum(APW, N_TASK - a0)
    zero = jnp.zeros((L,), jnp.float32)
    iota = lax.iota(jnp.int32, L)
    c2 = [c2_v[pl.ds(16 * j, L)] for j in range(NCH)]

    sems = ((sem0, sem2), (sem1, sem3))
    CK = 4 * DEG
    HK = CK // 2
    ncks = nv // 4
    pltpu.async_copy(abar_hbm.at[src_v.at[pl.ds(0, HK)]],
                     g_v.at[0, pl.ds(0, HK)], sems[0][0])
    pltpu.async_copy(abar_hbm.at[src_v.at[pl.ds(HK, HK)]],
                     g_v.at[0, pl.ds(HK, HK)], sems[0][1])

    def inner(ii, carry):
        par, c, ss, ss2 = carry
        i = c * 4 + ii
        b = [b_v[i, pl.ds(16 * j, L)] for j in range(NCH)]
        qs = plsc.load_gather(q_v, [jnp.full((L,), i, jnp.int32)])
        for grp in range(DEG // L):
            sv = src_v[pl.ds(i * DEG + grp * L, L)]
            svec = zero
            for k in range(L):
                e = grp * L + k
                acc = zero
                for j in range(NCH):
                    u = g_v[par, ii * DEG + e, pl.ds(16 * j, L)] + b[j]
                    acc = acc + c2[j] * jnp.abs(u)
                svec = jnp.where(iota == k, jnp.sum(acc), svec)
            svec = svec + plsc.load_gather(p_v, [sv]) + qs
            fg = plsc.load_gather(f_v, [sv])
            ss = ss + svec
            ss2 = ss2 + svec * svec
            sbuf[pl.ds(i * DEG + grp * L, L)] = jnp.where(
                fg > 0.5, jnp.float32(jnp.nan), svec)
        return par, c, ss, ss2

    def body(i2, carry):
        for par in (0, 1):
            c = i2 * 2 + par
            for h in (0, 1):
                pltpu.make_async_copy(
                    abar_hbm.at[pl.ds(0, HK)],
                    g_v.at[par, pl.ds(h * HK, HK)], sems[par][h]).wait()
            nxt = jnp.minimum(c + 1, ncks - 1)
            for h in (0, 1):
                pltpu.async_copy(
                    abar_hbm.at[src_v.at[pl.ds(nxt * CK + h * HK, HK)]],
                    g_v.at[par ^ 1, pl.ds(h * HK, HK)], sems[par ^ 1][h])
            _, _, ss, ss2 = lax.fori_loop(
                0, 4, inner, (jnp.int32(par), jnp.int32(c)) + carry)
            carry = (ss, ss2)
        return carry

    ss, ss2 = lax.fori_loop(0, ncks // 2, body, (zero, zero))
    for h in (0, 1):
        pltpu.make_async_copy(abar_hbm.at[pl.ds(0, HK)],
                              g_v.at[0, pl.ds(h * HK, HK)], sems[0][h]).wait()
    ovec = jnp.where(iota == 0, jnp.sum(ss), 0.0)
    ovec = jnp.where(iota == 1, jnp.sum(ss2), ovec)
    o_v[...] = ovec
    pltpu.sync_copy(o_v, p2_hbm.at[pl.ds(w * 16, 16)])
    pltpu.sync_copy(sbuf, s_hbm.at[pl.ds(a0 * DEG, EPW)])


def _final_body(s_ref, p2_ref, g2, b2, out_ref):
    p2 = p2_ref[...]
    ssum = jnp.sum(p2[:, 0])
    ssq = jnp.sum(p2[:, 1])
    mean = ssum / E
    var = ssq / E - mean * mean
    inv = g2[...] * lax.rsqrt(var + EPS)
    bias = b2[...] - mean * inv
    sv = s_ref[0:N_AG, :]
    vals = sv * inv + bias
    out_ref[...] = jnp.where(jnp.isnan(sv), -jnp.inf, vals)


_final = pl.pallas_call(
    _final_body,
    out_shape=jax.ShapeDtypeStruct((N_AG, DEG), jnp.float32),
)


@jax.jit
def kernel(nf, edge_index, node_type, W1, gamma1, beta1, W2, gamma2, beta2):
    nf_t = nf[:N_TASK]
    nf_a = nf[N_TASK:]
    w1a = W1[:D]
    w1b = W1[D:]
    src = edge_index[0].astype(jnp.int32)
    src_pad = jnp.pad(src, (0, E_PAD - E))

    A, B = _proj(nf_t, nf_a, w1a, w1b)
    part1 = _pass1(A, B, src_pad).reshape(NW, 2 * D)

    abar, bbar, p, q, c2, fnode = _fold(
        part1, A, B, gamma1, beta1, W2, node_type[:N_TASK].astype(jnp.int32))

    s_pad, part2 = _pass2(abar, bbar, src_pad, p, fnode, q, c2)
    part2 = part2.reshape(NW, 16)
    s = s_pad.reshape(N_AG_PAD, DEG)
    return _final(s, part2, gamma2, beta2)
